# Initial kernel scaffold; baseline (speedup 1.0000x reference)
import jax
import jax.numpy as jnp
from jax.experimental import pallas as pl


def kernel(x, positions, token_emb):
    raise NotImplementedError("write your pallas kernel here")


# speedup vs baseline: 1.6611x; 1.6611x over previous
import jax
import jax.numpy as jnp
from jax import lax
from jax.experimental import pallas as pl
from jax.experimental.pallas import tpu as pltpu
from jax.experimental.pallas import tpu_sc as plsc

_NUM_TOKENS = 1000
_D = 128
_DP = 131

_NC = 2
_NS = 16
_NW = _NC * _NS

_B = 4096 * 50
_BPW = _B // _NW
_CH = 400
_NCHUNK = _BPW // _CH


def _make_kernel():
    mesh = plsc.VectorSubcoreMesh(core_axis_name="c", subcore_axis_name="s")

    def body(tab_hbm, idx_hbm, pos_hbm, out_hbm, idx_v, stage_v, sem):
        wid = lax.axis_index("s") * _NC + lax.axis_index("c")
        wbase = wid * _BPW

        def step(i, carry):
            base = wbase + i * _CH
            pltpu.sync_copy(idx_hbm.at[pl.ds(base, _CH)], idx_v)
            pltpu.async_copy(
                tab_hbm.at[idx_v], stage_v.at[:, pl.ds(0, _D)], sem
            ).wait()
            pltpu.sync_copy(
                pos_hbm.at[pl.ds(base, _CH)], stage_v.at[:, pl.ds(_D, 3)]
            )
            pltpu.sync_copy(stage_v, out_hbm.at[pl.ds(base, _CH)])
            return carry

        lax.fori_loop(0, _NCHUNK, step, 0)

    return pl.kernel(
        body,
        out_type=jax.ShapeDtypeStruct((_B, _DP), jnp.float32),
        mesh=mesh,
        scratch_types=[
            pltpu.VMEM((_CH,), jnp.int32),
            pltpu.VMEM((_CH, _DP), jnp.float32),
            pltpu.SemaphoreType.DMA,
        ],
    )


_sc_kernel = _make_kernel()


@jax.jit
def kernel(x, positions, token_emb):
    m, a = x.shape
    x_flat = x.reshape(_B).astype(jnp.int32)
    pos_flat = positions.reshape(_B, 3)
    out = _sc_kernel(token_emb, x_flat, pos_flat)
    return out.reshape(m, a, _DP)


# speedup vs baseline: 1.7014x; 1.0243x over previous
import jax
import jax.numpy as jnp
from jax import lax
from jax.experimental import pallas as pl
from jax.experimental.pallas import tpu as pltpu
from jax.experimental.pallas import tpu_sc as plsc

_D = 128
_DP = 131

_NC = 2
_NS = 16
_NW = _NC * _NS

_B = 4096 * 50
_BPW = _B // _NW
_CH = 80
_NBUF = 4
_NCHUNK = _BPW // _CH
_NGROUP = _NCHUNK // _NBUF


def _make_kernel():
    mesh = plsc.VectorSubcoreMesh(core_axis_name="c", subcore_axis_name="s")

    def body(tab_hbm, idx_hbm, pos_hbm, out_hbm,
             idx_v, stages, gsems, psems, osems):
        wid = lax.axis_index("s") * _NC + lax.axis_index("c")
        wbase = wid * _BPW

        pltpu.sync_copy(idx_hbm.at[pl.ds(wbase, _BPW)], idx_v)

        def issue(b, i):
            g = pltpu.async_copy(
                tab_hbm.at[idx_v.at[pl.ds(i * _CH, _CH)]],
                stages[b].at[:, pl.ds(0, _D)],
                gsems[b],
            )
            p = pltpu.async_copy(
                pos_hbm.at[pl.ds(wbase + i * _CH, _CH)],
                stages[b].at[:, pl.ds(_D, 3)],
                psems[b],
            )
            return g, p

        def write_out(b, i, g, p):
            g.wait()
            p.wait()
            return pltpu.async_copy(
                stages[b],
                out_hbm.at[pl.ds(wbase + i * _CH, _CH)],
                osems[b],
            )

        def wait_out(b):
            pltpu.make_async_copy(
                stages[b], out_hbm.at[pl.ds(wbase, _CH)], osems[b]
            ).wait()

        descs = [issue(b, b) for b in range(_NBUF)]
        for b in range(_NBUF):
            write_out(b, b, *descs[b])

        def grp(g, carry):
            descs = []
            for b in range(_NBUF):
                i = g * _NBUF + b
                wait_out(b)
                descs.append(issue(b, i))
            for b in range(_NBUF):
                i = g * _NBUF + b
                write_out(b, i, *descs[b])
            return carry

        lax.fori_loop(1, _NGROUP, grp, 0)

        for b in range(_NBUF):
            wait_out(b)

    return pl.kernel(
        body,
        out_type=jax.ShapeDtypeStruct((_B, _DP), jnp.float32),
        mesh=mesh,
        scratch_types=[
            pltpu.VMEM((_BPW,), jnp.int32),
            [pltpu.VMEM((_CH, _DP), jnp.float32) for _ in range(_NBUF)],
            [pltpu.SemaphoreType.DMA for _ in range(_NBUF)],
            [pltpu.SemaphoreType.DMA for _ in range(_NBUF)],
            [pltpu.SemaphoreType.DMA for _ in range(_NBUF)],
        ],
    )


_sc_kernel = _make_kernel()


@jax.jit
def kernel(x, positions, token_emb):
    m, a = x.shape
    x_flat = x.reshape(_B).astype(jnp.int32)
    pos_flat = positions.reshape(_B, 3)
    out = _sc_kernel(token_emb, x_flat, pos_flat)
    return out.reshape(m, a, _DP)


# speedup vs baseline: 2.2845x; 1.3427x over previous
import jax
import jax.numpy as jnp
from jax import lax
from jax.experimental import pallas as pl
from jax.experimental.pallas import tpu as pltpu
from jax.experimental.pallas import tpu_sc as plsc

_M = 4096
_A = 50
_D = 128
_DP = 131

_NC = 2
_NS = 16
_NW = _NC * _NS

_MPW = _M // _NW
_NBUF = 4
_NGROUP = _MPW // _NBUF


def _make_kernel():
    mesh = plsc.VectorSubcoreMesh(core_axis_name="c", subcore_axis_name="s")

    def body(tab_hbm, x_hbm, pos_hbm, out_hbm,
             idx_v, stages, gsems, psems, osems):
        wid = lax.axis_index("s") * _NC + lax.axis_index("c")
        wmol = wid * _MPW

        pltpu.sync_copy(x_hbm.at[pl.ds(wmol, _MPW)], idx_v)

        def issue(b, i):
            g = pltpu.async_copy(
                tab_hbm.at[idx_v.at[i]],
                stages[b].at[:, pl.ds(0, _D)],
                gsems[b],
            )
            p = pltpu.async_copy(
                pos_hbm.at[wmol + i],
                stages[b].at[:, pl.ds(_D, 3)],
                psems[b],
            )
            return g, p

        def write_out(b, i, g, p):
            g.wait()
            p.wait()
            return pltpu.async_copy(
                stages[b], out_hbm.at[wmol + i], osems[b]
            )

        def wait_out(b):
            pltpu.make_async_copy(
                stages[b], out_hbm.at[wmol], osems[b]
            ).wait()

        descs = [issue(b, b) for b in range(_NBUF)]
        for b in range(_NBUF):
            write_out(b, b, *descs[b])

        def grp(g, carry):
            descs = []
            for b in range(_NBUF):
                wait_out(b)
                descs.append(issue(b, g * _NBUF + b))
            for b in range(_NBUF):
                write_out(b, g * _NBUF + b, *descs[b])
            return carry

        lax.fori_loop(1, _NGROUP, grp, 0)

        for b in range(_NBUF):
            wait_out(b)

    return pl.kernel(
        body,
        out_type=jax.ShapeDtypeStruct((_M, _A, _DP), jnp.float32),
        mesh=mesh,
        scratch_types=[
            pltpu.VMEM((_MPW, _A), jnp.int32),
            [pltpu.VMEM((_A, _DP), jnp.float32) for _ in range(_NBUF)],
            [pltpu.SemaphoreType.DMA for _ in range(_NBUF)],
            [pltpu.SemaphoreType.DMA for _ in range(_NBUF)],
            [pltpu.SemaphoreType.DMA for _ in range(_NBUF)],
        ],
    )


_sc_kernel = _make_kernel()


@jax.jit
def kernel(x, positions, token_emb):
    return _sc_kernel(token_emb, x.astype(jnp.int32), positions)
